# initial kernel scaffold (unmeasured)
import jax
import jax.numpy as jnp
from jax import lax
from jax.experimental import pallas as pl
from jax.experimental.pallas import tpu as pltpu


def kernel(
    x,
):
    def body(*refs):
        pass

    out_shape = jax.ShapeDtypeStruct(..., jnp.float32)
    return pl.pallas_call(body, out_shape=out_shape)(...)


# baseline (device time: 450202 ns/iter reference)
import jax
import jax.numpy as jnp
from jax import lax
from jax.experimental import pallas as pl
from jax.experimental.pallas import tpu as pltpu

M = 16384
N = 2048
NOUT = 1024
NCHUNK = 8
CM = M // NCHUNK


def kernel(x):
    def body(x_hbm, out_hbm, stage_mine, stage_peer, send_buf, recv_buf,
             res_buf, load_sems, send_sems, recv_sems, store_sem):
        my_x = lax.axis_index("x")
        my_y = lax.axis_index("y")
        peer = (1 - my_x, my_y)

        my_col = my_x * NOUT
        peer_col = (1 - my_x) * NOUT

        barrier_sem = pltpu.get_barrier_semaphore()
        pl.semaphore_signal(barrier_sem, inc=1, device_id=peer,
                            device_id_type=pl.DeviceIdType.MESH)
        pl.semaphore_wait(barrier_sem, 1)

        for c in range(NCHUNK):
            r0 = c * CM
            ld_p = pltpu.make_async_copy(
                x_hbm.at[0, pl.ds(r0, CM), pl.ds(peer_col, NOUT)],
                stage_peer, load_sems.at[0])
            ld_m = pltpu.make_async_copy(
                x_hbm.at[0, pl.ds(r0, CM), pl.ds(my_col, NOUT)],
                stage_mine, load_sems.at[1])
            ld_p.start()
            ld_m.start()
            ld_p.wait()

            send_buf[...] = stage_peer[...].astype(jnp.bfloat16)
            rdma = pltpu.make_async_remote_copy(
                src_ref=send_buf,
                dst_ref=recv_buf.at[c],
                send_sem=send_sems.at[c],
                recv_sem=recv_sems.at[c],
                device_id=peer,
                device_id_type=pl.DeviceIdType.MESH,
            )
            rdma.start()
            rdma.wait()

            ld_m.wait()
            res_buf[...] = (
                stage_mine[...] + recv_buf[c].astype(jnp.float32)
            ).astype(jnp.bfloat16)
            st = pltpu.make_async_copy(
                res_buf, out_hbm.at[pl.ds(r0, CM), :], store_sem)
            st.start()
            st.wait()

    return pl.pallas_call(
        body,
        out_shape=jax.ShapeDtypeStruct((M, NOUT), jnp.bfloat16),
        in_specs=[pl.BlockSpec(memory_space=pl.ANY)],
        out_specs=pl.BlockSpec(memory_space=pl.ANY),
        scratch_shapes=[
            pltpu.VMEM((CM, NOUT), jnp.float32),
            pltpu.VMEM((CM, NOUT), jnp.float32),
            pltpu.VMEM((CM, NOUT), jnp.bfloat16),
            pltpu.VMEM((NCHUNK, CM, NOUT), jnp.bfloat16),
            pltpu.VMEM((CM, NOUT), jnp.bfloat16),
            pltpu.SemaphoreType.DMA((2,)),
            pltpu.SemaphoreType.DMA((NCHUNK,)),
            pltpu.SemaphoreType.DMA((NCHUNK,)),
            pltpu.SemaphoreType.DMA,
        ],
        compiler_params=pltpu.CompilerParams(
            collective_id=0, vmem_limit_bytes=100 * 1024 * 1024),
    )(x)


# device time: 251720 ns/iter; 1.7885x vs baseline; 1.7885x over previous
import jax
import jax.numpy as jnp
from jax import lax
from jax.experimental import pallas as pl
from jax.experimental.pallas import tpu as pltpu

M = 16384
N = 2048
NOUT = 1024
HALF_M = M // 2
NC = 8
CM = HALF_M // NC
NSLOT = 4


def kernel(x):
    def body(x_hbm, out_hbm, stage_mine, stage_peer, xsend, xrecv, res,
             yrecv, load_sems_m, load_sems_p, xsend_sems, xrecv_sems,
             ysend_sems, yrecv_sems, store_sems_loc, store_sems_fwd):
        my_x = lax.axis_index("x")
        my_y = lax.axis_index("y")
        xpeer = (1 - my_x, my_y)
        ypeer = (my_x, 1 - my_y)

        my_col = my_x * NOUT
        peer_col = (1 - my_x) * NOUT
        my_base = my_y * HALF_M
        other_base = (1 - my_y) * HALF_M

        barrier_sem = pltpu.get_barrier_semaphore()
        for nbr in (xpeer, ypeer):
            pl.semaphore_signal(barrier_sem, inc=1, device_id=nbr,
                                device_id_type=pl.DeviceIdType.MESH)
        pl.semaphore_wait(barrier_sem, 2)

        def make_load(c, slot):
            lm = pltpu.make_async_copy(
                x_hbm.at[0, pl.ds(my_base + c * CM, CM), pl.ds(my_col, NOUT)],
                stage_mine.at[slot], load_sems_m.at[slot])
            lp = pltpu.make_async_copy(
                x_hbm.at[0, pl.ds(my_base + c * CM, CM), pl.ds(peer_col, NOUT)],
                stage_peer.at[slot], load_sems_p.at[slot])
            return lm, lp

        def make_store_fwd(c):
            return pltpu.make_async_copy(
                yrecv.at[c % NSLOT],
                out_hbm.at[pl.ds(other_base + c * CM, CM), :],
                store_sems_fwd.at[c % NSLOT])

        loads = [None] * NC
        loads[0] = make_load(0, 0)
        loads[0][0].start()
        loads[0][1].start()

        xrdmas = [None] * NC
        yrdmas = [None] * NC
        stores_loc = [None] * NC
        stores_fwd = [None] * NC

        for c in range(NC):
            s2 = c % 2
            s4 = c % NSLOT
            if c + 1 < NC:
                loads[c + 1] = make_load(c + 1, (c + 1) % 2)
                loads[c + 1][0].start()
                loads[c + 1][1].start()

            loads[c][1].wait()
            if c >= 2:
                xrdmas[c - 2].wait_send()
            xsend[s2] = stage_peer[s2].astype(jnp.bfloat16)
            xrdmas[c] = pltpu.make_async_remote_copy(
                src_ref=xsend.at[s2],
                dst_ref=xrecv.at[s4],
                send_sem=xsend_sems.at[s2],
                recv_sem=xrecv_sems.at[s4],
                device_id=xpeer,
                device_id_type=pl.DeviceIdType.MESH,
            )
            xrdmas[c].start()

            xrdmas[c].wait_recv()
            loads[c][0].wait()
            if c >= NSLOT:
                yrdmas[c - NSLOT].wait_send()
                stores_loc[c - NSLOT].wait()
            res[s4] = (stage_mine[s2]
                       + xrecv[s4].astype(jnp.float32)).astype(jnp.bfloat16)

            stores_loc[c] = pltpu.make_async_copy(
                res.at[s4], out_hbm.at[pl.ds(my_base + c * CM, CM), :],
                store_sems_loc.at[s4])
            stores_loc[c].start()
            if c >= 3:
                stores_fwd[c - 3].wait()
            yrdmas[c] = pltpu.make_async_remote_copy(
                src_ref=res.at[s4],
                dst_ref=yrecv.at[s4],
                send_sem=ysend_sems.at[s4],
                recv_sem=yrecv_sems.at[s4],
                device_id=ypeer,
                device_id_type=pl.DeviceIdType.MESH,
            )
            yrdmas[c].start()

            if c >= 1:
                yrdmas[c - 1].wait_recv()
                stores_fwd[c - 1] = make_store_fwd(c - 1)
                stores_fwd[c - 1].start()

        yrdmas[NC - 1].wait_recv()
        stores_fwd[NC - 1] = make_store_fwd(NC - 1)
        stores_fwd[NC - 1].start()

        xrdmas[NC - 2].wait_send()
        xrdmas[NC - 1].wait_send()
        for c in range(NC - NSLOT, NC):
            yrdmas[c].wait_send()
            stores_loc[c].wait()
        for c in range(NC - 3, NC):
            stores_fwd[c].wait()

    return pl.pallas_call(
        body,
        out_shape=jax.ShapeDtypeStruct((M, NOUT), jnp.bfloat16),
        in_specs=[pl.BlockSpec(memory_space=pl.ANY)],
        out_specs=pl.BlockSpec(memory_space=pl.ANY),
        scratch_shapes=[
            pltpu.VMEM((2, CM, NOUT), jnp.float32),
            pltpu.VMEM((2, CM, NOUT), jnp.float32),
            pltpu.VMEM((2, CM, NOUT), jnp.bfloat16),
            pltpu.VMEM((NSLOT, CM, NOUT), jnp.bfloat16),
            pltpu.VMEM((NSLOT, CM, NOUT), jnp.bfloat16),
            pltpu.VMEM((NSLOT, CM, NOUT), jnp.bfloat16),
            pltpu.SemaphoreType.DMA((2,)),
            pltpu.SemaphoreType.DMA((2,)),
            pltpu.SemaphoreType.DMA((2,)),
            pltpu.SemaphoreType.DMA((NSLOT,)),
            pltpu.SemaphoreType.DMA((NSLOT,)),
            pltpu.SemaphoreType.DMA((NSLOT,)),
            pltpu.SemaphoreType.DMA((NSLOT,)),
            pltpu.SemaphoreType.DMA((NSLOT,)),
        ],
        compiler_params=pltpu.CompilerParams(
            collective_id=0, vmem_limit_bytes=60 * 1024 * 1024),
    )(x)


# device time: 235249 ns/iter; 1.9137x vs baseline; 1.0700x over previous
import jax
import jax.numpy as jnp
from jax import lax
from jax.experimental import pallas as pl
from jax.experimental.pallas import tpu as pltpu

M = 16384
N = 2048
NOUT = 1024
HALF_M = M // 2
NC = 8
CM = HALF_M // NC
NSLOT = 4


def kernel(x):
    def body(x_hbm, out_hbm, stage_mine, stage_peer, xsend, xrecv, res,
             yrecv, load_sems_m, load_sems_p, xsend_sems, xrecv_sems,
             ysend_sems, yrecv_sems, store_sems_loc, store_sems_fwd):
        my_x = lax.axis_index("x")
        my_y = lax.axis_index("y")
        xpeer = (1 - my_x, my_y)
        ypeer = (my_x, 1 - my_y)

        my_col = my_x * NOUT
        peer_col = (1 - my_x) * NOUT
        my_base = my_y * HALF_M
        other_base = (1 - my_y) * HALF_M

        barrier_sem = pltpu.get_barrier_semaphore()
        for nbr in (xpeer, ypeer):
            pl.semaphore_signal(barrier_sem, inc=1, device_id=nbr,
                                device_id_type=pl.DeviceIdType.MESH)
        pl.semaphore_wait(barrier_sem, 2)

        def make_load(c):
            lm = pltpu.make_async_copy(
                x_hbm.at[0, pl.ds(my_base + c * CM, CM), pl.ds(my_col, NOUT)],
                stage_mine.at[c % NSLOT], load_sems_m.at[c % NSLOT])
            lp = pltpu.make_async_copy(
                x_hbm.at[0, pl.ds(my_base + c * CM, CM), pl.ds(peer_col, NOUT)],
                stage_peer.at[c % 2], load_sems_p.at[c % 2])
            return lm, lp

        def make_store_fwd(c):
            return pltpu.make_async_copy(
                yrecv.at[c % NSLOT],
                out_hbm.at[pl.ds(other_base + c * CM, CM), :],
                store_sems_fwd.at[c % NSLOT])

        loads = [None] * NC
        xrdmas = [None] * NC
        yrdmas = [None] * NC
        stores_loc = [None] * NC
        stores_fwd = [None] * NC

        loads[0] = make_load(0)
        loads[0][0].start()
        loads[0][1].start()

        def process(k):
            s4 = k % NSLOT
            xrdmas[k].wait_recv()
            loads[k][0].wait()
            if k >= NSLOT:
                yrdmas[k - NSLOT].wait_send()
                stores_loc[k - NSLOT].wait()
            res[s4] = (stage_mine[s4]
                       + xrecv[s4].astype(jnp.float32)).astype(jnp.bfloat16)

            stores_loc[k] = pltpu.make_async_copy(
                res.at[s4], out_hbm.at[pl.ds(my_base + k * CM, CM), :],
                store_sems_loc.at[s4])
            stores_loc[k].start()
            if k >= 3:
                stores_fwd[k - 3].wait()
            yrdmas[k] = pltpu.make_async_remote_copy(
                src_ref=res.at[s4],
                dst_ref=yrecv.at[s4],
                send_sem=ysend_sems.at[s4],
                recv_sem=yrecv_sems.at[s4],
                device_id=ypeer,
                device_id_type=pl.DeviceIdType.MESH,
            )
            yrdmas[k].start()

            if k >= 1:
                yrdmas[k - 1].wait_recv()
                stores_fwd[k - 1] = make_store_fwd(k - 1)
                stores_fwd[k - 1].start()

        for c in range(NC):
            if c + 1 < NC:
                loads[c + 1] = make_load(c + 1)
                loads[c + 1][0].start()
                loads[c + 1][1].start()

            loads[c][1].wait()
            if c >= 2:
                xrdmas[c - 2].wait_send()
            xsend[c % 2] = stage_peer[c % 2].astype(jnp.bfloat16)
            xrdmas[c] = pltpu.make_async_remote_copy(
                src_ref=xsend.at[c % 2],
                dst_ref=xrecv.at[c % NSLOT],
                send_sem=xsend_sems.at[c % 2],
                recv_sem=xrecv_sems.at[c % NSLOT],
                device_id=xpeer,
                device_id_type=pl.DeviceIdType.MESH,
            )
            xrdmas[c].start()

            if c >= 1:
                process(c - 1)

        process(NC - 1)

        yrdmas[NC - 1].wait_recv()
        stores_fwd[NC - 1] = make_store_fwd(NC - 1)
        stores_fwd[NC - 1].start()

        xrdmas[NC - 2].wait_send()
        xrdmas[NC - 1].wait_send()
        for c in range(NC - NSLOT, NC):
            yrdmas[c].wait_send()
            stores_loc[c].wait()
        for c in range(NC - 3, NC):
            stores_fwd[c].wait()

    return pl.pallas_call(
        body,
        out_shape=jax.ShapeDtypeStruct((M, NOUT), jnp.bfloat16),
        in_specs=[pl.BlockSpec(memory_space=pl.ANY)],
        out_specs=pl.BlockSpec(memory_space=pl.ANY),
        scratch_shapes=[
            pltpu.VMEM((NSLOT, CM, NOUT), jnp.float32),
            pltpu.VMEM((2, CM, NOUT), jnp.float32),
            pltpu.VMEM((2, CM, NOUT), jnp.bfloat16),
            pltpu.VMEM((NSLOT, CM, NOUT), jnp.bfloat16),
            pltpu.VMEM((NSLOT, CM, NOUT), jnp.bfloat16),
            pltpu.VMEM((NSLOT, CM, NOUT), jnp.bfloat16),
            pltpu.SemaphoreType.DMA((NSLOT,)),
            pltpu.SemaphoreType.DMA((2,)),
            pltpu.SemaphoreType.DMA((2,)),
            pltpu.SemaphoreType.DMA((NSLOT,)),
            pltpu.SemaphoreType.DMA((NSLOT,)),
            pltpu.SemaphoreType.DMA((NSLOT,)),
            pltpu.SemaphoreType.DMA((NSLOT,)),
            pltpu.SemaphoreType.DMA((NSLOT,)),
        ],
        compiler_params=pltpu.CompilerParams(
            collective_id=0, vmem_limit_bytes=60 * 1024 * 1024),
    )(x)


# device time: 222280 ns/iter; 2.0254x vs baseline; 1.0583x over previous
import jax
import jax.numpy as jnp
from jax import lax
from jax.experimental import pallas as pl
from jax.experimental.pallas import tpu as pltpu

M = 16384
N = 2048
NOUT = 1024
HALF_M = M // 2
NC = 16
CM = HALF_M // NC
NSLOT = 6
XS = 4


def kernel(x):
    def body(x_hbm, out_hbm, stage_mine, stage_peer, xsend, xrecv, res,
             yrecv, load_sems_m, load_sems_p, xsend_sems, xrecv_sems,
             ysend_sems, yrecv_sems, store_sems_loc, store_sems_fwd):
        my_x = lax.axis_index("x")
        my_y = lax.axis_index("y")
        xpeer = (1 - my_x, my_y)
        ypeer = (my_x, 1 - my_y)

        my_col = my_x * NOUT
        peer_col = (1 - my_x) * NOUT
        my_base = my_y * HALF_M
        other_base = (1 - my_y) * HALF_M

        barrier_sem = pltpu.get_barrier_semaphore()
        for nbr in (xpeer, ypeer):
            pl.semaphore_signal(barrier_sem, inc=1, device_id=nbr,
                                device_id_type=pl.DeviceIdType.MESH)
        pl.semaphore_wait(barrier_sem, 2)

        def make_load(c):
            lm = pltpu.make_async_copy(
                x_hbm.at[0, pl.ds(my_base + c * CM, CM), pl.ds(my_col, NOUT)],
                stage_mine.at[c % NSLOT], load_sems_m.at[c % NSLOT])
            lp = pltpu.make_async_copy(
                x_hbm.at[0, pl.ds(my_base + c * CM, CM), pl.ds(peer_col, NOUT)],
                stage_peer.at[c % 2], load_sems_p.at[c % 2])
            return lm, lp

        def make_store_fwd(c):
            return pltpu.make_async_copy(
                yrecv.at[c % NSLOT],
                out_hbm.at[pl.ds(other_base + c * CM, CM), :],
                store_sems_fwd.at[c % NSLOT])

        loads = [None] * NC
        xrdmas = [None] * NC
        yrdmas = [None] * NC
        stores_loc = [None] * NC
        stores_fwd = [None] * NC

        loads[0] = make_load(0)
        loads[0][0].start()
        loads[0][1].start()

        def process(k):
            s4 = k % NSLOT
            xrdmas[k].wait_recv()
            loads[k][0].wait()
            if k >= NSLOT:
                yrdmas[k - NSLOT].wait_send()
                stores_loc[k - NSLOT].wait()
            res[s4] = (stage_mine[s4]
                       + xrecv[s4].astype(jnp.float32)).astype(jnp.bfloat16)

            stores_loc[k] = pltpu.make_async_copy(
                res.at[s4], out_hbm.at[pl.ds(my_base + k * CM, CM), :],
                store_sems_loc.at[s4])
            stores_loc[k].start()
            if k >= NSLOT - 1:
                stores_fwd[k - (NSLOT - 1)].wait()
            yrdmas[k] = pltpu.make_async_remote_copy(
                src_ref=res.at[s4],
                dst_ref=yrecv.at[s4],
                send_sem=ysend_sems.at[s4],
                recv_sem=yrecv_sems.at[s4],
                device_id=ypeer,
                device_id_type=pl.DeviceIdType.MESH,
            )
            yrdmas[k].start()

            if k >= 1:
                yrdmas[k - 1].wait_recv()
                stores_fwd[k - 1] = make_store_fwd(k - 1)
                stores_fwd[k - 1].start()

        for c in range(NC):
            if c + 1 < NC:
                loads[c + 1] = make_load(c + 1)
                loads[c + 1][0].start()
                loads[c + 1][1].start()

            loads[c][1].wait()
            if c >= XS:
                xrdmas[c - XS].wait_send()
            xsend[c % XS] = stage_peer[c % 2].astype(jnp.bfloat16)
            xrdmas[c] = pltpu.make_async_remote_copy(
                src_ref=xsend.at[c % XS],
                dst_ref=xrecv.at[c % NSLOT],
                send_sem=xsend_sems.at[c % XS],
                recv_sem=xrecv_sems.at[c % NSLOT],
                device_id=xpeer,
                device_id_type=pl.DeviceIdType.MESH,
            )
            xrdmas[c].start()

            if c >= 1:
                process(c - 1)

        process(NC - 1)

        yrdmas[NC - 1].wait_recv()
        stores_fwd[NC - 1] = make_store_fwd(NC - 1)
        stores_fwd[NC - 1].start()

        for c in range(NC - XS, NC):
            xrdmas[c].wait_send()
        for c in range(NC - NSLOT, NC):
            yrdmas[c].wait_send()
            stores_loc[c].wait()
        for c in range(NC - (NSLOT - 1), NC):
            stores_fwd[c].wait()

    return pl.pallas_call(
        body,
        out_shape=jax.ShapeDtypeStruct((M, NOUT), jnp.bfloat16),
        in_specs=[pl.BlockSpec(memory_space=pl.ANY)],
        out_specs=pl.BlockSpec(memory_space=pl.ANY),
        scratch_shapes=[
            pltpu.VMEM((NSLOT, CM, NOUT), jnp.float32),
            pltpu.VMEM((2, CM, NOUT), jnp.float32),
            pltpu.VMEM((XS, CM, NOUT), jnp.bfloat16),
            pltpu.VMEM((NSLOT, CM, NOUT), jnp.bfloat16),
            pltpu.VMEM((NSLOT, CM, NOUT), jnp.bfloat16),
            pltpu.VMEM((NSLOT, CM, NOUT), jnp.bfloat16),
            pltpu.SemaphoreType.DMA((NSLOT,)),
            pltpu.SemaphoreType.DMA((2,)),
            pltpu.SemaphoreType.DMA((XS,)),
            pltpu.SemaphoreType.DMA((NSLOT,)),
            pltpu.SemaphoreType.DMA((NSLOT,)),
            pltpu.SemaphoreType.DMA((NSLOT,)),
            pltpu.SemaphoreType.DMA((NSLOT,)),
            pltpu.SemaphoreType.DMA((NSLOT,)),
        ],
        compiler_params=pltpu.CompilerParams(
            collective_id=0, vmem_limit_bytes=60 * 1024 * 1024),
    )(x)


# device time: 222093 ns/iter; 2.0271x vs baseline; 1.0008x over previous
import jax
import jax.numpy as jnp
from jax import lax
from jax.experimental import pallas as pl
from jax.experimental.pallas import tpu as pltpu

M = 16384
N = 2048
NOUT = 1024
HALF_M = M // 2
NC = 16
CM = HALF_M // NC
NSLOT = 6
XS = 4


def kernel(x):
    def body(x_hbm, out_hbm, stage_mine, stage_peer, xsend, xrecv, res,
             yrecv, load_sems_m, load_sems_p, xsend_sems, xrecv_sems,
             ysend_sems, yrecv_sems, store_sems_loc, store_sems_fwd):
        my_x = lax.axis_index("x")
        my_y = lax.axis_index("y")
        xpeer = (1 - my_x, my_y)
        ypeer = (my_x, 1 - my_y)

        my_col = my_x * NOUT
        peer_col = (1 - my_x) * NOUT
        my_base = my_y * HALF_M
        other_base = (1 - my_y) * HALF_M

        def make_load(c):
            lm = pltpu.make_async_copy(
                x_hbm.at[0, pl.ds(my_base + c * CM, CM), pl.ds(my_col, NOUT)],
                stage_mine.at[c % NSLOT], load_sems_m.at[c % NSLOT])
            lp = pltpu.make_async_copy(
                x_hbm.at[0, pl.ds(my_base + c * CM, CM), pl.ds(peer_col, NOUT)],
                stage_peer.at[c % 2], load_sems_p.at[c % 2])
            return lm, lp

        def make_store_fwd(c):
            return pltpu.make_async_copy(
                yrecv.at[c % NSLOT],
                out_hbm.at[pl.ds(other_base + c * CM, CM), :],
                store_sems_fwd.at[c % NSLOT])

        loads = [None] * NC
        xrdmas = [None] * NC
        yrdmas = [None] * NC
        stores_loc = [None] * NC
        stores_fwd = [None] * NC

        for c in (0, 1):
            loads[c] = make_load(c)
            loads[c][0].start()
            loads[c][1].start()

        barrier_sem = pltpu.get_barrier_semaphore()
        for nbr in (xpeer, ypeer):
            pl.semaphore_signal(barrier_sem, inc=1, device_id=nbr,
                                device_id_type=pl.DeviceIdType.MESH)
        pl.semaphore_wait(barrier_sem, 2)

        def process(k):
            s4 = k % NSLOT
            xrdmas[k].wait_recv()
            loads[k][0].wait()
            if k >= NSLOT:
                yrdmas[k - NSLOT].wait_send()
                stores_loc[k - NSLOT].wait()
            res[s4] = (stage_mine[s4]
                       + xrecv[s4].astype(jnp.float32)).astype(jnp.bfloat16)

            stores_loc[k] = pltpu.make_async_copy(
                res.at[s4], out_hbm.at[pl.ds(my_base + k * CM, CM), :],
                store_sems_loc.at[s4])
            stores_loc[k].start()
            if k >= NSLOT - 1:
                stores_fwd[k - (NSLOT - 1)].wait()
            yrdmas[k] = pltpu.make_async_remote_copy(
                src_ref=res.at[s4],
                dst_ref=yrecv.at[s4],
                send_sem=ysend_sems.at[s4],
                recv_sem=yrecv_sems.at[s4],
                device_id=ypeer,
                device_id_type=pl.DeviceIdType.MESH,
            )
            yrdmas[k].start()

            if k >= 1:
                yrdmas[k - 1].wait_recv()
                stores_fwd[k - 1] = make_store_fwd(k - 1)
                stores_fwd[k - 1].start()

        for c in range(NC):
            if c + 2 < NC:
                loads[c + 2] = make_load(c + 2)
                loads[c + 2][0].start()
                loads[c + 2][1].start()

            loads[c][1].wait()
            if c >= XS:
                xrdmas[c - XS].wait_send()
            xsend[c % XS] = stage_peer[c % 2].astype(jnp.bfloat16)
            xrdmas[c] = pltpu.make_async_remote_copy(
                src_ref=xsend.at[c % XS],
                dst_ref=xrecv.at[c % NSLOT],
                send_sem=xsend_sems.at[c % XS],
                recv_sem=xrecv_sems.at[c % NSLOT],
                device_id=xpeer,
                device_id_type=pl.DeviceIdType.MESH,
            )
            xrdmas[c].start()

            if c >= 2:
                process(c - 2)

        process(NC - 2)
        process(NC - 1)

        yrdmas[NC - 1].wait_recv()
        stores_fwd[NC - 1] = make_store_fwd(NC - 1)
        stores_fwd[NC - 1].start()

        for c in range(NC - XS, NC):
            xrdmas[c].wait_send()
        for c in range(NC - NSLOT, NC):
            yrdmas[c].wait_send()
            stores_loc[c].wait()
        for c in range(NC - (NSLOT - 1), NC):
            stores_fwd[c].wait()

    return pl.pallas_call(
        body,
        out_shape=jax.ShapeDtypeStruct((M, NOUT), jnp.bfloat16),
        in_specs=[pl.BlockSpec(memory_space=pl.ANY)],
        out_specs=pl.BlockSpec(memory_space=pl.ANY),
        scratch_shapes=[
            pltpu.VMEM((NSLOT, CM, NOUT), jnp.float32),
            pltpu.VMEM((2, CM, NOUT), jnp.float32),
            pltpu.VMEM((XS, CM, NOUT), jnp.bfloat16),
            pltpu.VMEM((NSLOT, CM, NOUT), jnp.bfloat16),
            pltpu.VMEM((NSLOT, CM, NOUT), jnp.bfloat16),
            pltpu.VMEM((NSLOT, CM, NOUT), jnp.bfloat16),
            pltpu.SemaphoreType.DMA((NSLOT,)),
            pltpu.SemaphoreType.DMA((2,)),
            pltpu.SemaphoreType.DMA((XS,)),
            pltpu.SemaphoreType.DMA((NSLOT,)),
            pltpu.SemaphoreType.DMA((NSLOT,)),
            pltpu.SemaphoreType.DMA((NSLOT,)),
            pltpu.SemaphoreType.DMA((NSLOT,)),
            pltpu.SemaphoreType.DMA((NSLOT,)),
        ],
        compiler_params=pltpu.CompilerParams(
            collective_id=0, vmem_limit_bytes=60 * 1024 * 1024),
    )(x)


# device time: 212127 ns/iter; 2.1223x vs baseline; 1.0470x over previous
import jax
import jax.numpy as jnp
from jax import lax
from jax.experimental import pallas as pl
from jax.experimental.pallas import tpu as pltpu

M = 16384
N = 2048
NOUT = 1024
HALF_M = M // 2
NC = 64
CM = HALF_M // NC
NSLOT = 6
XS = 4
PS = 4


def kernel(x):
    def body(x_hbm, out_hbm, stage_mine, stage_peer, xsend, xrecv, res,
             yrecv, load_sems_m, load_sems_p, xsend_sems, xrecv_sems,
             ysend_sems, yrecv_sems, store_sems_loc, store_sems_fwd):
        my_x = lax.axis_index("x")
        my_y = lax.axis_index("y")
        xpeer = (1 - my_x, my_y)
        ypeer = (my_x, 1 - my_y)

        my_col = my_x * NOUT
        peer_col = (1 - my_x) * NOUT
        my_base = my_y * HALF_M
        other_base = (1 - my_y) * HALF_M

        def make_load(c):
            lm = pltpu.make_async_copy(
                x_hbm.at[0, pl.ds(my_base + c * CM, CM), pl.ds(my_col, NOUT)],
                stage_mine.at[c % NSLOT], load_sems_m.at[c % NSLOT])
            lp = pltpu.make_async_copy(
                x_hbm.at[0, pl.ds(my_base + c * CM, CM), pl.ds(peer_col, NOUT)],
                stage_peer.at[c % PS], load_sems_p.at[c % PS])
            return lm, lp

        def make_store_fwd(c):
            return pltpu.make_async_copy(
                yrecv.at[c % NSLOT],
                out_hbm.at[pl.ds(other_base + c * CM, CM), :],
                store_sems_fwd.at[c % NSLOT])

        loads = [None] * NC
        xrdmas = [None] * NC
        yrdmas = [None] * NC
        stores_loc = [None] * NC
        stores_fwd = [None] * NC

        for c in (0, 1):
            loads[c] = make_load(c)
            loads[c][0].start()
            loads[c][1].start()

        barrier_sem = pltpu.get_barrier_semaphore()
        for nbr in (xpeer, ypeer):
            pl.semaphore_signal(barrier_sem, inc=1, device_id=nbr,
                                device_id_type=pl.DeviceIdType.MESH)
        pl.semaphore_wait(barrier_sem, 2)

        def process(k):
            s4 = k % NSLOT
            xrdmas[k].wait_recv()
            loads[k][0].wait()
            if k >= NSLOT:
                yrdmas[k - NSLOT].wait_send()
                stores_loc[k - NSLOT].wait()
            res[s4] = (stage_mine[s4]
                       + xrecv[s4].astype(jnp.float32)).astype(jnp.bfloat16)

            stores_loc[k] = pltpu.make_async_copy(
                res.at[s4], out_hbm.at[pl.ds(my_base + k * CM, CM), :],
                store_sems_loc.at[s4])
            stores_loc[k].start()
            if k >= NSLOT - 1:
                stores_fwd[k - (NSLOT - 1)].wait()
            yrdmas[k] = pltpu.make_async_remote_copy(
                src_ref=res.at[s4],
                dst_ref=yrecv.at[s4],
                send_sem=ysend_sems.at[s4],
                recv_sem=yrecv_sems.at[s4],
                device_id=ypeer,
                device_id_type=pl.DeviceIdType.MESH,
            )
            yrdmas[k].start()

            if k >= 1:
                yrdmas[k - 1].wait_recv()
                stores_fwd[k - 1] = make_store_fwd(k - 1)
                stores_fwd[k - 1].start()

        for c in range(NC):
            if c + 2 < NC:
                loads[c + 2] = make_load(c + 2)
                loads[c + 2][0].start()
                loads[c + 2][1].start()

            loads[c][1].wait()
            if c >= XS:
                xrdmas[c - XS].wait_send()
            xsend[c % XS] = stage_peer[c % PS].astype(jnp.bfloat16)
            xrdmas[c] = pltpu.make_async_remote_copy(
                src_ref=xsend.at[c % XS],
                dst_ref=xrecv.at[c % NSLOT],
                send_sem=xsend_sems.at[c % XS],
                recv_sem=xrecv_sems.at[c % NSLOT],
                device_id=xpeer,
                device_id_type=pl.DeviceIdType.MESH,
            )
            xrdmas[c].start()

            if c >= 2:
                process(c - 2)

        process(NC - 2)
        process(NC - 1)

        yrdmas[NC - 1].wait_recv()
        stores_fwd[NC - 1] = make_store_fwd(NC - 1)
        stores_fwd[NC - 1].start()

        for c in range(NC - XS, NC):
            xrdmas[c].wait_send()
        for c in range(NC - NSLOT, NC):
            yrdmas[c].wait_send()
            stores_loc[c].wait()
        for c in range(NC - (NSLOT - 1), NC):
            stores_fwd[c].wait()

    return pl.pallas_call(
        body,
        out_shape=jax.ShapeDtypeStruct((M, NOUT), jnp.bfloat16),
        in_specs=[pl.BlockSpec(memory_space=pl.ANY)],
        out_specs=pl.BlockSpec(memory_space=pl.ANY),
        scratch_shapes=[
            pltpu.VMEM((NSLOT, CM, NOUT), jnp.float32),
            pltpu.VMEM((PS, CM, NOUT), jnp.float32),
            pltpu.VMEM((XS, CM, NOUT), jnp.bfloat16),
            pltpu.VMEM((NSLOT, CM, NOUT), jnp.bfloat16),
            pltpu.VMEM((NSLOT, CM, NOUT), jnp.bfloat16),
            pltpu.VMEM((NSLOT, CM, NOUT), jnp.bfloat16),
            pltpu.SemaphoreType.DMA((NSLOT,)),
            pltpu.SemaphoreType.DMA((PS,)),
            pltpu.SemaphoreType.DMA((XS,)),
            pltpu.SemaphoreType.DMA((NSLOT,)),
            pltpu.SemaphoreType.DMA((NSLOT,)),
            pltpu.SemaphoreType.DMA((NSLOT,)),
            pltpu.SemaphoreType.DMA((NSLOT,)),
            pltpu.SemaphoreType.DMA((NSLOT,)),
        ],
        compiler_params=pltpu.CompilerParams(
            collective_id=0, vmem_limit_bytes=60 * 1024 * 1024),
    )(x)
